# baseline (speedup 1.0000x reference)
import functools

import jax
import jax.numpy as jnp
from jax import lax
from jax.experimental import pallas as pl
from jax.experimental.pallas import tpu as pltpu
from jax.experimental.pallas import tpu_sc as plsc

B = 16384
D = 32
NC = 2
NS = 16
NW = NC * NS
B_PER_W = B // NW
IDX_CH = B_PER_W // 128


def _sc_gather(uids, iids, user_table, item_table):
  mesh = plsc.VectorSubcoreMesh(
      core_axis_name="c", subcore_axis_name="s", num_cores=NC, num_subcores=NS
  )

  @functools.partial(
      pl.kernel,
      out_type=[
          jax.ShapeDtypeStruct((B, D), jnp.float32),
          jax.ShapeDtypeStruct((B, D), jnp.float32),
      ],
      mesh=mesh,
      compiler_params=pltpu.CompilerParams(use_tc_tiling_on_sc=False),
      scratch_types=[
          pltpu.VMEM((IDX_CH, 128), jnp.int32),
          pltpu.VMEM((IDX_CH, 128), jnp.int32),
          pltpu.VMEM((B_PER_W, D), jnp.float32),
          pltpu.VMEM((B_PER_W, D), jnp.float32),
          pltpu.SemaphoreType.DMA,
      ],
  )
  def gather_kernel(uids_hbm, iids_hbm, utab_hbm, itab_hbm, out_u, out_i,
                    idxu_v, idxi_v, rows_u, rows_i, sem):
    wid = lax.axis_index("s") * NC + lax.axis_index("c")
    base = wid * B_PER_W
    pltpu.sync_copy(uids_hbm.at[wid], idxu_v)
    pltpu.sync_copy(iids_hbm.at[wid], idxi_v)
    copies = []
    for j in range(IDX_CH):
      copies.append(pltpu.async_copy(
          utab_hbm.at[idxu_v.at[j]], rows_u.at[pl.ds(j * 128, 128)], sem))
      copies.append(pltpu.async_copy(
          itab_hbm.at[idxi_v.at[j]], rows_i.at[pl.ds(j * 128, 128)], sem))
    for c in copies:
      c.wait()
    pltpu.sync_copy(rows_u, out_u.at[pl.ds(base, B_PER_W)])
    pltpu.sync_copy(rows_i, out_i.at[pl.ds(base, B_PER_W)])

  return gather_kernel(uids, iids, user_table, item_table)


def _mlp_body(u_ref, v_ref, w1u_ref, w1v_ref, b1_ref, w2_ref, b2_ref,
              w3_ref, b3_ref, o_ref):
  u = u_ref[...]
  v = v_ref[...]
  h = u @ w1u_ref[...] + v @ w1v_ref[...] + b1_ref[...]
  h = jnp.maximum(h, 0.0)
  h2 = jnp.maximum(h @ w2_ref[...] + b2_ref[...], 0.0)
  o_ref[...] = jnp.sum(h2 * w3_ref[...], axis=1) + b3_ref[...]


def _tc_mlp(emb_u, emb_i, w1u, w1v, b1, w2, b2, w3, b3, block_b=2048):
  grid = (B // block_b,)
  return pl.pallas_call(
      _mlp_body,
      grid=grid,
      in_specs=[
          pl.BlockSpec((block_b, D), lambda i: (i, 0)),
          pl.BlockSpec((block_b, D), lambda i: (i, 0)),
          pl.BlockSpec((D, 64), lambda i: (0, 0)),
          pl.BlockSpec((D, 64), lambda i: (0, 0)),
          pl.BlockSpec((64,), lambda i: (0,)),
          pl.BlockSpec((64, 16), lambda i: (0, 0)),
          pl.BlockSpec((16,), lambda i: (0,)),
          pl.BlockSpec((1, 16), lambda i: (0, 0)),
          pl.BlockSpec((1,), lambda i: (0,)),
      ],
      out_specs=pl.BlockSpec((block_b,), lambda i: (i,)),
      out_shape=jax.ShapeDtypeStruct((B,), jnp.float32),
  )(emb_u, emb_i, w1u, w1v, b1, w2, b2, w3, b3)


def kernel(user_ids, item_ids, user_table, item_table, W1, b1, W2, b2, W3, b3):
  uids = user_ids.astype(jnp.int32).reshape(NW, IDX_CH, 128)
  iids = item_ids.astype(jnp.int32).reshape(NW, IDX_CH, 128)
  emb_u, emb_i = _sc_gather(uids, iids, user_table, item_table)
  w1u = W1[:, :D].T
  w1v = W1[:, D:].T
  w2 = W2.T
  return _tc_mlp(emb_u, emb_i, w1u, w1v, b1, w2, b2, W3, b3)
